# baseline (speedup 1.0000x reference)
import functools

import jax
import jax.numpy as jnp
from jax import lax
from jax.experimental import pallas as pl
from jax.experimental.pallas import tpu as pltpu
from jax.experimental.pallas import tpu_sc as plsc

N = 10000
E = 320000
D = 128
NC = 2
NS = 16
NW = NC * NS
CHUNK = 128
CPT = 80
E_PAD = CHUNK * CPT * NW
NP = 10112
RPT = NP // NS
HCOL = 4
AT = E_PAD // CHUNK // NS
A0 = 80
A1 = AT - A0
SCPT = 16


@functools.cache
def _mesh():
    return plsc.VectorSubcoreMesh(core_axis_name="c", subcore_axis_name="s",
                                  num_cores=NC, num_subcores=NS)


def _worker_id():
    return lax.axis_index("s") * NC + lax.axis_index("c")


def _deg_body(src_hbm, dst_hbm, hist_out, src_v, dst_v, hs, hd):
    wid = _worker_id()
    pltpu.sync_copy(src_hbm.at[wid], src_v)
    pltpu.sync_copy(dst_hbm.at[wid], dst_v)

    zero16 = jnp.zeros((16,), jnp.float32)

    @pl.loop(0, HCOL * NP // 16)
    def _zero(i):
        hs[pl.ds(i * 16, 16)] = zero16
        hd[pl.ds(i * 16, 16)] = zero16

    lane = lax.iota(jnp.int32, 16)
    laneoff = (lane % HCOL) * NP
    ones = jnp.ones((16,), jnp.float32)
    masks = [(lane >= 4 * g) & (lane < 4 * (g + 1)) for g in range(4)]

    @pl.loop(0, CPT)
    def _hist(j):
        for k in range(CHUNK // 16):
            sv = src_v[j, pl.ds(k * 16, 16)] + laneoff
            dv = dst_v[j, pl.ds(k * 16, 16)] + laneoff
            for g in range(4):
                plsc.addupdate_scatter(hs, [sv], ones, mask=masks[g])
                plsc.addupdate_scatter(hd, [dv], ones, mask=masks[g])

    pltpu.sync_copy(hs, hist_out.at[wid, 0])
    pltpu.sync_copy(hd, hist_out.at[wid, 1])


@functools.cache
def _deg_call():
    return pl.kernel(
        _deg_body,
        out_type=jax.ShapeDtypeStruct((NW, 2, HCOL * NP), jnp.float32),
        mesh=_mesh(),
        scratch_types=[
            pltpu.VMEM((CPT, CHUNK), jnp.int32),
            pltpu.VMEM((CPT, CHUNK), jnp.int32),
            pltpu.VMEM((HCOL * NP,), jnp.float32),
            pltpu.VMEM((HCOL * NP,), jnp.float32),
        ],
        compiler_params=pltpu.CompilerParams(needs_layout_passes=False),
    )


def _agg_body(m_hbm, src_hbm, dst_hbm, acc_out, src_va, src_vb, dst_va,
              dst_vb, buf0, buf1, acc_sh, g0, g1):
    c = lax.axis_index("c")
    s = lax.axis_index("s")

    zero16 = jnp.zeros((16,), jnp.float32)

    @pl.loop(0, CHUNK)
    def _zero(i):
        for k in range(D // 16):
            buf0[i, pl.ds(k * 16, 16)] = zero16

    base = s * RPT
    for r in range(RPT // CHUNK):
        pltpu.sync_copy(buf0, acc_sh.at[pl.ds(base + r * CHUNK, CHUNK)])
    rem = RPT - (RPT // CHUNK) * CHUNK
    if rem:
        pltpu.sync_copy(buf0.at[pl.ds(0, rem)],
                        acc_sh.at[pl.ds(base + (RPT // CHUNK) * CHUNK, rem)])
    plsc.subcore_barrier()

    def emit_sections(chunk0, nsec):
        sbufs = (src_va, src_vb)
        dbufs = (dst_va, dst_vb)
        pltpu.sync_copy(src_hbm.at[s, pl.ds(chunk0, SCPT)], sbufs[0])
        pltpu.sync_copy(dst_hbm.at[s, pl.ds(chunk0, SCPT)], dbufs[0])
        pltpu.async_copy(m_hbm.at[sbufs[0].at[0]], buf0, g0)
        pltpu.async_copy(m_hbm.at[sbufs[0].at[1]], buf1, g1)
        for q in range(nsec):
            sv, dv = sbufs[q % 2], dbufs[q % 2]
            svn, dvn = sbufs[(q + 1) % 2], dbufs[(q + 1) % 2]
            if q + 1 < nsec:
                off = chunk0 + (q + 1) * SCPT
                pltpu.sync_copy(src_hbm.at[s, pl.ds(off, SCPT)], svn)
                pltpu.sync_copy(dst_hbm.at[s, pl.ds(off, SCPT)], dvn)

            @pl.loop(0, SCPT // 2 - 1)
            def _edges(t):
                j0 = t * 2
                pltpu.make_async_copy(m_hbm.at[sv.at[j0]], buf0, g0).wait()
                pltpu.sync_copy(buf0, acc_sh.at[dv.at[j0]], add=True)
                pltpu.async_copy(m_hbm.at[sv.at[j0 + 2]], buf0, g0)
                pltpu.make_async_copy(m_hbm.at[sv.at[j0 + 1]], buf1, g1).wait()
                pltpu.sync_copy(buf1, acc_sh.at[dv.at[j0 + 1]], add=True)
                pltpu.async_copy(m_hbm.at[sv.at[j0 + 3]], buf1, g1)

            pltpu.make_async_copy(m_hbm.at[sv.at[SCPT - 2]], buf0, g0).wait()
            pltpu.sync_copy(buf0, acc_sh.at[dv.at[SCPT - 2]], add=True)
            if q + 1 < nsec:
                pltpu.async_copy(m_hbm.at[svn.at[0]], buf0, g0)
            pltpu.make_async_copy(m_hbm.at[sv.at[SCPT - 1]], buf1, g1).wait()
            pltpu.sync_copy(buf1, acc_sh.at[dv.at[SCPT - 1]], add=True)
            if q + 1 < nsec:
                pltpu.async_copy(m_hbm.at[svn.at[1]], buf1, g1)

    @pl.when(c == 0)
    def _fast():
        emit_sections(0, A0 // SCPT)

    @pl.when(c == 1)
    def _slow():
        emit_sections(A0, A1 // SCPT)

    plsc.subcore_barrier()
    pltpu.sync_copy(acc_sh.at[pl.ds(base, RPT)],
                    acc_out.at[c, pl.ds(base, RPT)])


@functools.cache
def _agg_call():
    return pl.kernel(
        _agg_body,
        out_type=jax.ShapeDtypeStruct((NC, NP, D), jnp.float32),
        mesh=_mesh(),
        scratch_types=[
            pltpu.VMEM((SCPT, CHUNK), jnp.int32),
            pltpu.VMEM((SCPT, CHUNK), jnp.int32),
            pltpu.VMEM((SCPT, CHUNK), jnp.int32),
            pltpu.VMEM((SCPT, CHUNK), jnp.int32),
            pltpu.VMEM((CHUNK, D), jnp.float32),
            pltpu.VMEM((CHUNK, D), jnp.float32),
            pltpu.VMEM_SHARED((NP, D), jnp.float32),
            pltpu.SemaphoreType.DMA,
            pltpu.SemaphoreType.DMA,
        ],
        compiler_params=pltpu.CompilerParams(needs_layout_passes=False),
    )


def _tc1_body(hist_ref, h_ref, w1_ref, m_ref, dis_ref):
    hsum = jnp.sum(hist_ref[...], axis=0)
    deg = sum(hsum[:, g * NP:(g + 1) * NP] for g in range(HCOL))
    dis = lax.rsqrt(jnp.maximum(deg, 1.0))
    dis_t = jnp.transpose(dis)
    dis_ref[...] = dis_t
    xw = jnp.dot(h_ref[...], w1_ref[...], preferred_element_type=jnp.float32)
    m_ref[...] = jnp.concatenate(
        [xw * dis_t[:N, 0:1], jnp.zeros((NP - N, D), jnp.float32)], axis=0)


def _tc1(hist, h, w1):
    return pl.pallas_call(
        _tc1_body,
        out_shape=(jax.ShapeDtypeStruct((NP, D), jnp.float32),
                   jax.ShapeDtypeStruct((NP, 2), jnp.float32)),
    )(hist, h, w1)


def _tc2_body(acc_ref, dis_ref, b_ref, w_ref, m_ref):
    accs = acc_ref[0] + acc_ref[1]
    x = jnp.maximum(accs[:N] * dis_ref[:N, 1:2] + b_ref[...], 0.0)
    xw = jnp.dot(x, w_ref[...], preferred_element_type=jnp.float32)
    m_ref[...] = jnp.concatenate(
        [xw * dis_ref[:N, 0:1], jnp.zeros((NP - N, D), jnp.float32)], axis=0)


def _tc2(acc, dis, b, w):
    return pl.pallas_call(
        _tc2_body,
        out_shape=jax.ShapeDtypeStruct((NP, D), jnp.float32),
    )(acc, dis, b, w)


def _tc3_body(acc_ref, dis_ref, b_ref, wr_ref, br_ref, y_ref):
    accs = acc_ref[0] + acc_ref[1]
    x = jnp.maximum(accs[:N] * dis_ref[:N, 1:2] + b_ref[...], 0.0)
    hg = jnp.sum(x, axis=0, keepdims=True) * (1.0 / N)
    y = jnp.sum(hg * wr_ref[...]) + br_ref[0, 0]
    y_ref[...] = jnp.full((1, 1), 0.0, jnp.float32) + y


def _tc3(acc, dis, b, wr, br):
    return pl.pallas_call(
        _tc3_body,
        out_shape=jax.ShapeDtypeStruct((1, 1), jnp.float32),
    )(acc, dis, b, wr, br)


def kernel(h, edge_index, W1, b1, W2, b2, Wr, br):
    ei = edge_index.astype(jnp.int32)
    ei = jnp.pad(ei, ((0, 0), (0, E_PAD - E)), constant_values=N)
    src_t = ei[0].reshape(NW, CPT, CHUNK)
    dst_t = ei[1].reshape(NW, CPT, CHUNK)
    src_a = ei[0].reshape(NS, AT, CHUNK)
    dst_a = ei[1].reshape(NS, AT, CHUNK)

    hist = _deg_call()(src_t, dst_t)
    m1, dis = _tc1(hist, h, W1)
    acc1 = _agg_call()(m1, src_a, dst_a)
    m2 = _tc2(acc1, dis, b1.reshape(1, D), W2)
    acc2 = _agg_call()(m2, src_a, dst_a)
    y = _tc3(acc2, dis, b2.reshape(1, D), Wr.reshape(1, D),
             br.reshape(1, 1))
    return y


# speedup vs baseline: 2.5652x; 2.5652x over previous
import functools

import jax
import jax.numpy as jnp
from jax import lax
from jax.experimental import pallas as pl
from jax.experimental.pallas import tpu as pltpu
from jax.experimental.pallas import tpu_sc as plsc

N = 10000
E = 320000
D = 128
NC = 2
NS = 16
NW = NC * NS
CHUNK = 128
CPT = 80
E_PAD = CHUNK * CPT * NW
NP = 10112
RPT = NP // NS
HCOL = 4
AT = E_PAD // CHUNK // NS
A0 = 128
A1 = AT - A0
SCPT = 32


@functools.cache
def _mesh():
    return plsc.VectorSubcoreMesh(core_axis_name="c", subcore_axis_name="s",
                                  num_cores=NC, num_subcores=NS)


def _worker_id():
    return lax.axis_index("s") * NC + lax.axis_index("c")


def _deg_body(src_hbm, dst_hbm, hist_out, src_v, dst_v, hs, hd):
    wid = _worker_id()
    pltpu.sync_copy(src_hbm.at[wid], src_v)
    pltpu.sync_copy(dst_hbm.at[wid], dst_v)

    zero16 = jnp.zeros((16,), jnp.float32)

    @pl.loop(0, HCOL * NP // 16)
    def _zero(i):
        hs[pl.ds(i * 16, 16)] = zero16
        hd[pl.ds(i * 16, 16)] = zero16

    lane = lax.iota(jnp.int32, 16)
    laneoff = (lane % HCOL) * NP
    ones = jnp.ones((16,), jnp.float32)
    masks = [(lane >= 4 * g) & (lane < 4 * (g + 1)) for g in range(4)]

    @pl.loop(0, CPT)
    def _hist(j):
        for k in range(CHUNK // 16):
            sv = src_v[j, pl.ds(k * 16, 16)] + laneoff
            dv = dst_v[j, pl.ds(k * 16, 16)] + laneoff
            for g in range(4):
                plsc.addupdate_scatter(hs, [sv], ones, mask=masks[g])
                plsc.addupdate_scatter(hd, [dv], ones, mask=masks[g])

    pltpu.sync_copy(hs, hist_out.at[wid, 0])
    pltpu.sync_copy(hd, hist_out.at[wid, 1])


@functools.cache
def _deg_call():
    return pl.kernel(
        _deg_body,
        out_type=jax.ShapeDtypeStruct((NW, 2, HCOL * NP), jnp.float32),
        mesh=_mesh(),
        scratch_types=[
            pltpu.VMEM((CPT, CHUNK), jnp.int32),
            pltpu.VMEM((CPT, CHUNK), jnp.int32),
            pltpu.VMEM((HCOL * NP,), jnp.float32),
            pltpu.VMEM((HCOL * NP,), jnp.float32),
        ],
        compiler_params=pltpu.CompilerParams(needs_layout_passes=False),
    )


def _agg_body(m_hbm, src_hbm, dst_hbm, acc_out, src_va, src_vb, dst_va,
              dst_vb, buf0, buf1, acc_sh, g0, g1):
    c = lax.axis_index("c")
    s = lax.axis_index("s")

    zero16 = jnp.zeros((16,), jnp.float32)

    @pl.loop(0, CHUNK)
    def _zero(i):
        for k in range(D // 16):
            buf0[i, pl.ds(k * 16, 16)] = zero16

    base = s * RPT
    for r in range(RPT // CHUNK):
        pltpu.sync_copy(buf0, acc_sh.at[pl.ds(base + r * CHUNK, CHUNK)])
    rem = RPT - (RPT // CHUNK) * CHUNK
    if rem:
        pltpu.sync_copy(buf0.at[pl.ds(0, rem)],
                        acc_sh.at[pl.ds(base + (RPT // CHUNK) * CHUNK, rem)])
    plsc.subcore_barrier()

    def emit_sections(chunk0, nsec):
        sbufs = (src_va, src_vb)
        dbufs = (dst_va, dst_vb)
        pltpu.sync_copy(src_hbm.at[s, pl.ds(chunk0, SCPT)], sbufs[0])
        pltpu.sync_copy(dst_hbm.at[s, pl.ds(chunk0, SCPT)], dbufs[0])
        pltpu.async_copy(m_hbm.at[sbufs[0].at[0]], buf0, g0)
        pltpu.async_copy(m_hbm.at[sbufs[0].at[1]], buf1, g1)
        for q in range(nsec):
            sv, dv = sbufs[q % 2], dbufs[q % 2]
            svn, dvn = sbufs[(q + 1) % 2], dbufs[(q + 1) % 2]
            if q + 1 < nsec:
                off = chunk0 + (q + 1) * SCPT
                pltpu.sync_copy(src_hbm.at[s, pl.ds(off, SCPT)], svn)
                pltpu.sync_copy(dst_hbm.at[s, pl.ds(off, SCPT)], dvn)

            @pl.loop(0, SCPT // 2 - 1)
            def _edges(t):
                j0 = t * 2
                pltpu.make_async_copy(m_hbm.at[sv.at[j0]], buf0, g0).wait()
                pltpu.sync_copy(buf0, acc_sh.at[dv.at[j0]], add=True)
                pltpu.async_copy(m_hbm.at[sv.at[j0 + 2]], buf0, g0)
                pltpu.make_async_copy(m_hbm.at[sv.at[j0 + 1]], buf1, g1).wait()
                pltpu.sync_copy(buf1, acc_sh.at[dv.at[j0 + 1]], add=True)
                pltpu.async_copy(m_hbm.at[sv.at[j0 + 3]], buf1, g1)

            pltpu.make_async_copy(m_hbm.at[sv.at[SCPT - 2]], buf0, g0).wait()
            pltpu.sync_copy(buf0, acc_sh.at[dv.at[SCPT - 2]], add=True)
            if q + 1 < nsec:
                pltpu.async_copy(m_hbm.at[svn.at[0]], buf0, g0)
            pltpu.make_async_copy(m_hbm.at[sv.at[SCPT - 1]], buf1, g1).wait()
            pltpu.sync_copy(buf1, acc_sh.at[dv.at[SCPT - 1]], add=True)
            if q + 1 < nsec:
                pltpu.async_copy(m_hbm.at[svn.at[1]], buf1, g1)

    @pl.when(c == 0)
    def _fast():
        emit_sections(0, A0 // SCPT)

    @pl.when(c == 1)
    def _slow():
        emit_sections(A0, A1 // SCPT)

    plsc.subcore_barrier()
    pltpu.sync_copy(acc_sh.at[pl.ds(base, RPT)],
                    acc_out.at[c, pl.ds(base, RPT)])


@functools.cache
def _agg_call():
    return pl.kernel(
        _agg_body,
        out_type=jax.ShapeDtypeStruct((NC, NP, D), jnp.float32),
        mesh=_mesh(),
        scratch_types=[
            pltpu.VMEM((SCPT, CHUNK), jnp.int32),
            pltpu.VMEM((SCPT, CHUNK), jnp.int32),
            pltpu.VMEM((SCPT, CHUNK), jnp.int32),
            pltpu.VMEM((SCPT, CHUNK), jnp.int32),
            pltpu.VMEM((CHUNK, D), jnp.float32),
            pltpu.VMEM((CHUNK, D), jnp.float32),
            pltpu.VMEM_SHARED((NP, D), jnp.float32),
            pltpu.SemaphoreType.DMA,
            pltpu.SemaphoreType.DMA,
        ],
        compiler_params=pltpu.CompilerParams(needs_layout_passes=False),
    )


def _tc1_body(hist_ref, h_ref, w1_ref, m_ref, dis_ref):
    hsum = jnp.sum(hist_ref[...], axis=0)
    deg = sum(hsum[:, g * NP:(g + 1) * NP] for g in range(HCOL))
    dis = lax.rsqrt(jnp.maximum(deg, 1.0))
    dis_t = jnp.transpose(dis)
    dis_ref[...] = dis_t
    xw = jnp.dot(h_ref[...], w1_ref[...], preferred_element_type=jnp.float32)
    m_ref[...] = jnp.concatenate(
        [xw * dis_t[:N, 0:1], jnp.zeros((NP - N, D), jnp.float32)], axis=0)


def _tc1(hist, h, w1):
    return pl.pallas_call(
        _tc1_body,
        out_shape=(jax.ShapeDtypeStruct((NP, D), jnp.float32),
                   jax.ShapeDtypeStruct((NP, 2), jnp.float32)),
    )(hist, h, w1)


def _tc2_body(acc_ref, dis_ref, b_ref, w_ref, m_ref):
    accs = acc_ref[0] + acc_ref[1]
    x = jnp.maximum(accs[:N] * dis_ref[:N, 1:2] + b_ref[...], 0.0)
    xw = jnp.dot(x, w_ref[...], preferred_element_type=jnp.float32)
    m_ref[...] = jnp.concatenate(
        [xw * dis_ref[:N, 0:1], jnp.zeros((NP - N, D), jnp.float32)], axis=0)


def _tc2(acc, dis, b, w):
    return pl.pallas_call(
        _tc2_body,
        out_shape=jax.ShapeDtypeStruct((NP, D), jnp.float32),
    )(acc, dis, b, w)


def _tc3_body(acc_ref, dis_ref, b_ref, wr_ref, br_ref, y_ref):
    accs = acc_ref[0] + acc_ref[1]
    x = jnp.maximum(accs[:N] * dis_ref[:N, 1:2] + b_ref[...], 0.0)
    hg = jnp.sum(x, axis=0, keepdims=True) * (1.0 / N)
    y = jnp.sum(hg * wr_ref[...]) + br_ref[0, 0]
    y_ref[...] = jnp.full((1, 1), 0.0, jnp.float32) + y


def _tc3(acc, dis, b, wr, br):
    return pl.pallas_call(
        _tc3_body,
        out_shape=jax.ShapeDtypeStruct((1, 1), jnp.float32),
    )(acc, dis, b, wr, br)


def kernel(h, edge_index, W1, b1, W2, b2, Wr, br):
    ei = edge_index.astype(jnp.int32)
    pad = N + (jnp.arange(E_PAD - E, dtype=jnp.int32) % (NP - N))
    ei = jnp.concatenate([ei, jnp.stack([pad, pad])], axis=1)
    src_t = ei[0].reshape(NW, CPT, CHUNK)
    dst_t = ei[1].reshape(NW, CPT, CHUNK)
    src_a = ei[0].reshape(NS, AT, CHUNK)
    dst_a = ei[1].reshape(NS, AT, CHUNK)

    hist = _deg_call()(src_t, dst_t)
    m1, dis = _tc1(hist, h, W1)
    acc1 = _agg_call()(m1, src_a, dst_a)
    m2 = _tc2(acc1, dis, b1.reshape(1, D), W2)
    acc2 = _agg_call()(m2, src_a, dst_a)
    y = _tc3(acc2, dis, b2.reshape(1, D), Wr.reshape(1, D),
             br.reshape(1, 1))
    return y


# speedup vs baseline: 3.4324x; 1.3380x over previous
import functools

import jax
import jax.numpy as jnp
from jax import lax
from jax.experimental import pallas as pl
from jax.experimental.pallas import tpu as pltpu
from jax.experimental.pallas import tpu_sc as plsc

N = 10000
E = 320000
D = 128
NC = 2
NS = 16
NW = NC * NS
CHUNK = 128
CPT = 80
E_PAD = CHUNK * CPT * NW
NP = 10112
RPT = NP // NS
HCOL = 4
AT = E_PAD // CHUNK // NS
A0 = 80
A1 = AT - A0
SCPT = 16


@functools.cache
def _mesh():
    return plsc.VectorSubcoreMesh(core_axis_name="c", subcore_axis_name="s",
                                  num_cores=NC, num_subcores=NS)


def _worker_id():
    return lax.axis_index("s") * NC + lax.axis_index("c")


def _deg_body(src_hbm, dst_hbm, hist_out, src_v, dst_v, hs, hd):
    wid = _worker_id()
    pltpu.sync_copy(src_hbm.at[wid], src_v)
    pltpu.sync_copy(dst_hbm.at[wid], dst_v)

    zero16 = jnp.zeros((16,), jnp.float32)

    @pl.loop(0, HCOL * NP // 16)
    def _zero(i):
        hs[pl.ds(i * 16, 16)] = zero16
        hd[pl.ds(i * 16, 16)] = zero16

    lane = lax.iota(jnp.int32, 16)
    laneoff = (lane % HCOL) * NP
    ones = jnp.ones((16,), jnp.float32)
    masks = [(lane >= 4 * g) & (lane < 4 * (g + 1)) for g in range(4)]

    @pl.loop(0, CPT)
    def _hist(j):
        for k in range(CHUNK // 16):
            sv = src_v[j, pl.ds(k * 16, 16)] + laneoff
            dv = dst_v[j, pl.ds(k * 16, 16)] + laneoff
            for g in range(4):
                plsc.addupdate_scatter(hs, [sv], ones, mask=masks[g])
                plsc.addupdate_scatter(hd, [dv], ones, mask=masks[g])

    pltpu.sync_copy(hs, hist_out.at[wid, 0])
    pltpu.sync_copy(hd, hist_out.at[wid, 1])


@functools.cache
def _deg_call():
    return pl.kernel(
        _deg_body,
        out_type=jax.ShapeDtypeStruct((NW, 2, HCOL * NP), jnp.float32),
        mesh=_mesh(),
        scratch_types=[
            pltpu.VMEM((CPT, CHUNK), jnp.int32),
            pltpu.VMEM((CPT, CHUNK), jnp.int32),
            pltpu.VMEM((HCOL * NP,), jnp.float32),
            pltpu.VMEM((HCOL * NP,), jnp.float32),
        ],
        compiler_params=pltpu.CompilerParams(needs_layout_passes=False),
    )


def _agg_body(m_hbm, src_hbm, dst_hbm, acc_out, src_va, src_vb, dst_va,
              dst_vb, buf0, buf1, acc_sh, g0, g1):
    c = lax.axis_index("c")
    s = lax.axis_index("s")

    zero16 = jnp.zeros((16,), jnp.float32)

    @pl.loop(0, CHUNK)
    def _zero(i):
        for k in range(D // 16):
            buf0[i, pl.ds(k * 16, 16)] = zero16

    base = s * RPT
    for r in range(RPT // CHUNK):
        pltpu.sync_copy(buf0, acc_sh.at[pl.ds(base + r * CHUNK, CHUNK)])
    rem = RPT - (RPT // CHUNK) * CHUNK
    if rem:
        pltpu.sync_copy(buf0.at[pl.ds(0, rem)],
                        acc_sh.at[pl.ds(base + (RPT // CHUNK) * CHUNK, rem)])
    plsc.subcore_barrier()

    def emit_sections(chunk0, nsec):
        sbufs = (src_va, src_vb)
        dbufs = (dst_va, dst_vb)
        pltpu.sync_copy(src_hbm.at[s, pl.ds(chunk0, SCPT)], sbufs[0])
        pltpu.sync_copy(dst_hbm.at[s, pl.ds(chunk0, SCPT)], dbufs[0])
        pltpu.async_copy(m_hbm.at[sbufs[0].at[0]], buf0, g0)
        pltpu.async_copy(m_hbm.at[sbufs[0].at[1]], buf1, g1)
        for q in range(nsec):
            sv, dv = sbufs[q % 2], dbufs[q % 2]
            svn, dvn = sbufs[(q + 1) % 2], dbufs[(q + 1) % 2]
            if q + 1 < nsec:
                off = chunk0 + (q + 1) * SCPT
                pltpu.sync_copy(src_hbm.at[s, pl.ds(off, SCPT)], svn)
                pltpu.sync_copy(dst_hbm.at[s, pl.ds(off, SCPT)], dvn)

            @pl.loop(0, SCPT // 2 - 1)
            def _edges(t):
                j0 = t * 2
                pltpu.make_async_copy(m_hbm.at[sv.at[j0]], buf0, g0).wait()
                pltpu.sync_copy(buf0, acc_sh.at[dv.at[j0]], add=True)
                pltpu.async_copy(m_hbm.at[sv.at[j0 + 2]], buf0, g0)
                pltpu.make_async_copy(m_hbm.at[sv.at[j0 + 1]], buf1, g1).wait()
                pltpu.sync_copy(buf1, acc_sh.at[dv.at[j0 + 1]], add=True)
                pltpu.async_copy(m_hbm.at[sv.at[j0 + 3]], buf1, g1)

            pltpu.make_async_copy(m_hbm.at[sv.at[SCPT - 2]], buf0, g0).wait()
            pltpu.sync_copy(buf0, acc_sh.at[dv.at[SCPT - 2]], add=True)
            if q + 1 < nsec:
                pltpu.async_copy(m_hbm.at[svn.at[0]], buf0, g0)
            pltpu.make_async_copy(m_hbm.at[sv.at[SCPT - 1]], buf1, g1).wait()
            pltpu.sync_copy(buf1, acc_sh.at[dv.at[SCPT - 1]], add=True)
            if q + 1 < nsec:
                pltpu.async_copy(m_hbm.at[svn.at[1]], buf1, g1)

    @pl.when(c == 0)
    def _fast():
        emit_sections(0, A0 // SCPT)

    @pl.when(c == 1)
    def _slow():
        emit_sections(A0, A1 // SCPT)

    plsc.subcore_barrier()
    pltpu.sync_copy(acc_sh.at[pl.ds(base, RPT)],
                    acc_out.at[c, pl.ds(base, RPT)])


@functools.cache
def _agg_call():
    return pl.kernel(
        _agg_body,
        out_type=jax.ShapeDtypeStruct((NC, NP, D), jnp.float32),
        mesh=_mesh(),
        scratch_types=[
            pltpu.VMEM((SCPT, CHUNK), jnp.int32),
            pltpu.VMEM((SCPT, CHUNK), jnp.int32),
            pltpu.VMEM((SCPT, CHUNK), jnp.int32),
            pltpu.VMEM((SCPT, CHUNK), jnp.int32),
            pltpu.VMEM((CHUNK, D), jnp.float32),
            pltpu.VMEM((CHUNK, D), jnp.float32),
            pltpu.VMEM_SHARED((NP, D), jnp.float32),
            pltpu.SemaphoreType.DMA,
            pltpu.SemaphoreType.DMA,
        ],
        compiler_params=pltpu.CompilerParams(needs_layout_passes=False),
    )


def _tc1_body(hist_ref, h_ref, w1_ref, m_ref, dis_ref):
    hsum = jnp.sum(hist_ref[...], axis=0)
    deg = sum(hsum[:, g * NP:(g + 1) * NP] for g in range(HCOL))
    dis = lax.rsqrt(jnp.maximum(deg, 1.0))
    dis_t = jnp.transpose(dis)
    dis_ref[...] = dis_t
    xw = jnp.dot(h_ref[...], w1_ref[...], preferred_element_type=jnp.float32)
    m_ref[...] = jnp.concatenate(
        [xw * dis_t[:N, 0:1], jnp.zeros((NP - N, D), jnp.float32)], axis=0)


def _tc1(hist, h, w1):
    return pl.pallas_call(
        _tc1_body,
        out_shape=(jax.ShapeDtypeStruct((NP, D), jnp.float32),
                   jax.ShapeDtypeStruct((NP, 2), jnp.float32)),
    )(hist, h, w1)


def _tc2_body(acc_ref, dis_ref, b_ref, w_ref, m_ref):
    accs = acc_ref[0] + acc_ref[1]
    x = jnp.maximum(accs[:N] * dis_ref[:N, 1:2] + b_ref[...], 0.0)
    xw = jnp.dot(x, w_ref[...], preferred_element_type=jnp.float32)
    m_ref[...] = jnp.concatenate(
        [xw * dis_ref[:N, 0:1], jnp.zeros((NP - N, D), jnp.float32)], axis=0)


def _tc2(acc, dis, b, w):
    return pl.pallas_call(
        _tc2_body,
        out_shape=jax.ShapeDtypeStruct((NP, D), jnp.float32),
    )(acc, dis, b, w)


def _tc3_body(acc_ref, dis_ref, b_ref, wr_ref, br_ref, y_ref):
    accs = acc_ref[0] + acc_ref[1]
    x = jnp.maximum(accs[:N] * dis_ref[:N, 1:2] + b_ref[...], 0.0)
    hg = jnp.sum(x, axis=0, keepdims=True) * (1.0 / N)
    y = jnp.sum(hg * wr_ref[...]) + br_ref[0, 0]
    y_ref[...] = jnp.full((1, 1), 0.0, jnp.float32) + y


def _tc3(acc, dis, b, wr, br):
    return pl.pallas_call(
        _tc3_body,
        out_shape=jax.ShapeDtypeStruct((1, 1), jnp.float32),
    )(acc, dis, b, wr, br)


def kernel(h, edge_index, W1, b1, W2, b2, Wr, br):
    ei = edge_index.astype(jnp.int32)
    pad = N + (jnp.arange(E_PAD - E, dtype=jnp.int32) % (NP - N))
    ei = jnp.concatenate([ei, jnp.stack([pad, pad])], axis=1)
    src_t = ei[0].reshape(NW, CPT, CHUNK)
    dst_t = ei[1].reshape(NW, CPT, CHUNK)
    src_a = ei[0].reshape(NS, AT, CHUNK)
    dst_a = ei[1].reshape(NS, AT, CHUNK)

    hist = _deg_call()(src_t, dst_t)
    m1, dis = _tc1(hist, h, W1)
    acc1 = _agg_call()(m1, src_a, dst_a)
    m2 = _tc2(acc1, dis, b1.reshape(1, D), W2)
    acc2 = _agg_call()(m2, src_a, dst_a)
    y = _tc3(acc2, dis, b2.reshape(1, D), Wr.reshape(1, D),
             br.reshape(1, 1))
    return y
